# baseline (speedup 1.0000x reference)
import functools

import jax
import jax.numpy as jnp
from jax import lax
from jax.experimental import pallas as pl
from jax.experimental.pallas import tpu as pltpu
from jax.experimental.pallas import tpu_sc as plsc

VOC = 100000
EMB = 128
BATCH = 1024

_info = plsc.get_sparse_core_info()
_NC, _NS = 1, _info.num_subcores
_NW = _NC * _NS
_BPW = BATCH // _NW


@functools.partial(
    pl.kernel,
    mesh=plsc.VectorSubcoreMesh(
        core_axis_name="c", subcore_axis_name="s", num_cores=1
    ),
    out_type=jax.ShapeDtypeStruct((BATCH, EMB), jnp.float32),
    scratch_types=[
        pltpu.VMEM((_BPW,), jnp.int32),
        pltpu.VMEM((_BPW, EMB), jnp.float32),
        pltpu.SemaphoreType.DMA,
    ],
)
def _sc_gather(table_hbm, idx_hbm, out_hbm, idx_v, rows_v, sem):
    wid = lax.axis_index("s") * _NC + lax.axis_index("c")
    base = wid * _BPW
    pltpu.sync_copy(idx_hbm.at[pl.ds(base, _BPW)], idx_v)
    pltpu.async_copy(table_hbm.at[idx_v], rows_v, sem).wait()
    pltpu.sync_copy(rows_v, out_hbm.at[pl.ds(base, _BPW)])


_TILE_V = 2048
_NT = (VOC + _TILE_V - 1) // _TILE_V


def _mm_body(wt_ref, emb_ref, b_ref, o_ref):
    bias_col = jnp.transpose(b_ref[...].reshape(1, _TILE_V))
    prod = jax.lax.dot_general(
        wt_ref[...], emb_ref[...],
        dimension_numbers=(((1,), (1,)), ((), ())),
        preferred_element_type=jnp.float32,
    )
    o_ref[...] = prod + bias_col


def _tc_matmul_t(Wt, emb, brow):
    return pl.pallas_call(
        _mm_body,
        grid=(_NT,),
        in_specs=[
            pl.BlockSpec((_TILE_V, EMB), lambda j: (j, 0)),
            pl.BlockSpec((BATCH, EMB), lambda j: (0, 0)),
            pl.BlockSpec((_TILE_V,), lambda j: (j,)),
        ],
        out_specs=pl.BlockSpec((_TILE_V, BATCH), lambda j: (j, 0)),
        out_shape=jax.ShapeDtypeStruct((VOC, BATCH), jnp.float32),
        compiler_params=pltpu.CompilerParams(
            dimension_semantics=("arbitrary",),
        ),
    )(Wt, emb, brow)


def kernel(X, emb_table, W, b):
    emb = _sc_gather(emb_table, X.astype(jnp.int32))
    out_t = _tc_matmul_t(W.T, emb, b)
    return out_t.T


# speedup vs baseline: 1.0147x; 1.0147x over previous
import functools

import jax
import jax.numpy as jnp
from jax import lax
from jax.experimental import pallas as pl
from jax.experimental.pallas import tpu as pltpu
from jax.experimental.pallas import tpu_sc as plsc

VOC = 100000
EMB = 128
BATCH = 1024

_info = plsc.get_sparse_core_info()
_NC, _NS = 1, _info.num_subcores
_NW = _NC * _NS
_BPW = BATCH // _NW


_H = _BPW // 2


@functools.partial(
    pl.kernel,
    mesh=plsc.VectorSubcoreMesh(
        core_axis_name="c", subcore_axis_name="s", num_cores=1
    ),
    out_type=jax.ShapeDtypeStruct((BATCH, EMB), jnp.float32),
    scratch_types=[
        pltpu.VMEM((_H,), jnp.int32),
        pltpu.VMEM((_H,), jnp.int32),
        pltpu.VMEM((_H, EMB), jnp.float32),
        pltpu.VMEM((_H, EMB), jnp.float32),
        pltpu.SemaphoreType.DMA,
        pltpu.SemaphoreType.DMA,
        pltpu.SemaphoreType.DMA,
    ],
)
def _sc_gather(table_hbm, idx_hbm, out_hbm, idx0, idx1, r0, r1, s0, s1, sw):
    wid = lax.axis_index("s") * _NC + lax.axis_index("c")
    base = wid * _BPW
    pltpu.sync_copy(idx_hbm.at[pl.ds(base, _H)], idx0)
    g0 = pltpu.async_copy(table_hbm.at[idx0], r0, s0)
    pltpu.sync_copy(idx_hbm.at[pl.ds(base + _H, _H)], idx1)
    g1 = pltpu.async_copy(table_hbm.at[idx1], r1, s1)
    g0.wait()
    w0 = pltpu.async_copy(r0, out_hbm.at[pl.ds(base, _H)], sw)
    g1.wait()
    pltpu.sync_copy(r1, out_hbm.at[pl.ds(base + _H, _H)])
    w0.wait()


_TILE_V = 4096
_NT = (VOC + _TILE_V - 1) // _TILE_V


def _mm_body(wt_ref, emb_ref, b_ref, o_ref):
    bias_col = jnp.transpose(b_ref[...].reshape(1, _TILE_V))
    prod = jax.lax.dot_general(
        wt_ref[...], emb_ref[...],
        dimension_numbers=(((1,), (1,)), ((), ())),
        preferred_element_type=jnp.float32,
    )
    o_ref[...] = prod + bias_col


def _tc_matmul_t(Wt, emb, brow):
    return pl.pallas_call(
        _mm_body,
        grid=(_NT,),
        in_specs=[
            pl.BlockSpec((_TILE_V, EMB), lambda j: (j, 0)),
            pl.BlockSpec((BATCH, EMB), lambda j: (0, 0)),
            pl.BlockSpec((_TILE_V,), lambda j: (j,)),
        ],
        out_specs=pl.BlockSpec((_TILE_V, BATCH), lambda j: (j, 0)),
        out_shape=jax.ShapeDtypeStruct((VOC, BATCH), jnp.float32),
        compiler_params=pltpu.CompilerParams(
            dimension_semantics=("arbitrary",),
        ),
    )(Wt, emb, brow)


def kernel(X, emb_table, W, b):
    emb = _sc_gather(emb_table, X.astype(jnp.int32))
    out_t = _tc_matmul_t(W.T, emb, b)
    return out_t.T


# speedup vs baseline: 1.0160x; 1.0013x over previous
import functools

import jax
import jax.numpy as jnp
from jax import lax
from jax.experimental import pallas as pl
from jax.experimental.pallas import tpu as pltpu
from jax.experimental.pallas import tpu_sc as plsc

VOC = 100000
EMB = 128
BATCH = 1024

_info = plsc.get_sparse_core_info()
_NC, _NS = 1, _info.num_subcores
_NW = _NC * _NS
_BPW = BATCH // _NW


@functools.partial(
    pl.kernel,
    mesh=plsc.VectorSubcoreMesh(
        core_axis_name="c", subcore_axis_name="s", num_cores=1
    ),
    out_type=jax.ShapeDtypeStruct((BATCH, EMB), jnp.float32),
    scratch_types=[
        pltpu.VMEM((_BPW,), jnp.int32),
        pltpu.VMEM((_BPW, EMB), jnp.float32),
        pltpu.SemaphoreType.DMA,
    ],
)
def _sc_gather(table_hbm, idx_hbm, out_hbm, idx_v, rows_v, sem):
    wid = lax.axis_index("s") * _NC + lax.axis_index("c")
    base = wid * _BPW
    pltpu.sync_copy(idx_hbm.at[pl.ds(base, _BPW)], idx_v)
    pltpu.async_copy(table_hbm.at[idx_v], rows_v, sem).wait()
    pltpu.sync_copy(rows_v, out_hbm.at[pl.ds(base, _BPW)])


_TILE_V = 4096
_NT = (VOC + _TILE_V - 1) // _TILE_V


def _mm_body(wt_ref, emb_ref, b_ref, o_ref):
    bias_col = jnp.transpose(b_ref[...].reshape(1, _TILE_V))
    prod = jax.lax.dot_general(
        wt_ref[...], emb_ref[...],
        dimension_numbers=(((1,), (1,)), ((), ())),
        preferred_element_type=jnp.float32,
    )
    o_ref[...] = prod + bias_col


def _tc_matmul_t(Wt, emb, brow):
    return pl.pallas_call(
        _mm_body,
        grid=(_NT,),
        in_specs=[
            pl.BlockSpec((_TILE_V, EMB), lambda j: (j, 0)),
            pl.BlockSpec((BATCH, EMB), lambda j: (0, 0)),
            pl.BlockSpec((_TILE_V,), lambda j: (j,)),
        ],
        out_specs=pl.BlockSpec((_TILE_V, BATCH), lambda j: (j, 0)),
        out_shape=jax.ShapeDtypeStruct((VOC, BATCH), jnp.float32),
        compiler_params=pltpu.CompilerParams(
            dimension_semantics=("arbitrary",),
        ),
    )(Wt, emb, brow)


def kernel(X, emb_table, W, b):
    emb = _sc_gather(emb_table, X.astype(jnp.int32))
    out_t = _tc_matmul_t(W.T, emb, b)
    return out_t.T


# speedup vs baseline: 1.0271x; 1.0108x over previous
import functools

import jax
import jax.numpy as jnp
from jax import lax
from jax.experimental import pallas as pl
from jax.experimental.pallas import tpu as pltpu
from jax.experimental.pallas import tpu_sc as plsc

VOC = 100000
EMB = 128
BATCH = 1024

_info = plsc.get_sparse_core_info()
_NC, _NS = 1, _info.num_subcores
_NW = _NC * _NS
_BPW = BATCH // _NW


@functools.partial(
    pl.kernel,
    mesh=plsc.VectorSubcoreMesh(
        core_axis_name="c", subcore_axis_name="s", num_cores=1
    ),
    out_type=jax.ShapeDtypeStruct((BATCH, EMB), jnp.float32),
    scratch_types=[
        pltpu.VMEM((_BPW,), jnp.int32),
        pltpu.VMEM((_BPW, EMB), jnp.float32),
        pltpu.SemaphoreType.DMA,
    ],
)
def _sc_gather(table_hbm, idx_hbm, out_hbm, idx_v, rows_v, sem):
    wid = lax.axis_index("s") * _NC + lax.axis_index("c")
    base = wid * _BPW
    pltpu.sync_copy(idx_hbm.at[pl.ds(base, _BPW)], idx_v)
    pltpu.async_copy(table_hbm.at[idx_v], rows_v, sem).wait()
    pltpu.sync_copy(rows_v, out_hbm.at[pl.ds(base, _BPW)])


_TILE_V = 4096
_NT = (VOC + _TILE_V - 1) // _TILE_V


def _mm_body(wt_ref, emb_ref, b_ref, o_ref):
    bias_col = jnp.transpose(b_ref[...].reshape(1, _TILE_V))
    prod = jax.lax.dot_general(
        wt_ref[...], emb_ref[...],
        dimension_numbers=(((1,), (1,)), ((), ())),
        preferred_element_type=jnp.float32,
    )
    o_ref[...] = prod + bias_col


def _tc_matmul_t(Wt, emb, brow):
    return pl.pallas_call(
        _mm_body,
        grid=(_NT,),
        in_specs=[
            pl.BlockSpec((_TILE_V, EMB), lambda j: (j, 0)),
            pl.BlockSpec((BATCH, EMB), lambda j: (0, 0)),
            pl.BlockSpec((_TILE_V,), lambda j: (j,)),
        ],
        out_specs=pl.BlockSpec((_TILE_V, BATCH), lambda j: (j, 0)),
        out_shape=jax.ShapeDtypeStruct((VOC, BATCH), jnp.float32),
        compiler_params=pltpu.CompilerParams(
            dimension_semantics=("parallel",),
        ),
    )(Wt, emb, brow)


def kernel(X, emb_table, W, b):
    emb = _sc_gather(emb_table, X.astype(jnp.int32))
    out_t = _tc_matmul_t(W.T, emb, b)
    return out_t.T
